# Initial kernel scaffold; baseline (speedup 1.0000x reference)
import jax
import jax.numpy as jnp
from jax.experimental import pallas as pl


def kernel(x, edge_index, edge_type, w_comp, basis, h_bias):
    raise NotImplementedError("write your pallas kernel here")


# speedup vs baseline: 5.3360x; 5.3360x over previous
import jax
import jax.numpy as jnp
from jax import lax
from jax.experimental import pallas as pl
from jax.experimental.pallas import tpu as pltpu
from jax.experimental.pallas import tpu_sc as plsc

N = 10000
E = 320000
R = 8
NB_BASES = 4
D = 128

NC = 2
NS = 16

CW = 16
NCHUNKS = D // CW + 1
CPS = 5
XROWS = NCHUNKS * N

GB = 128
EPT = 20480
EPAD = EPT * NS
BPP = EPT // GB
CH = 2048
NSUB = EPT // CH

ROWS_SC = 80128
RPT = ROWS_SC // NS
TRASH = R * N

_f32 = jnp.float32
_i32 = jnp.int32


def _sc_body(src_hbm, dst_hbm, typ_hbm, xflat_hbm, zacc_hbm,
             acc_out,
             gl_v, sl2_v, edst_v, etyp_v, rows0_v, rows1_v,
             acc_sh, gsem0, gsem1):
  c = lax.axis_index("c")
  s = lax.axis_index("s")
  ebase = s * EPT

  pltpu.sync_copy(src_hbm.at[pl.ds(ebase, EPT)], gl_v)

  def pre_chunk(q, _):
    pltpu.sync_copy(dst_hbm.at[pl.ds(ebase + q * CH, CH)], edst_v)
    pltpu.sync_copy(typ_hbm.at[pl.ds(ebase + q * CH, CH)], etyp_v)

    def pre_row(b, _):
      row = q * (CH // GB) + b
      for k in range(GB // 16):
        off = b * GB + k * 16
        dv = edst_v[pl.ds(off, 16)]
        tv = etyp_v[pl.ds(off, 16)]
        sl2_v[row, pl.ds(k * 16, 16)] = tv * N + dv
      return 0

    lax.fori_loop(0, CH // GB, pre_row, 0)
    return 0

  lax.fori_loop(0, NSUB, pre_chunk, 0)

  first_off = c * CPS * N

  def shift0(i, _):
    gl_v[pl.ds(i * 16, 16)] = gl_v[pl.ds(i * 16, 16)] + first_off
    return 0

  lax.fori_loop(0, EPT // 16, shift0, 0)

  def one_pass(p, _):
    @pl.when(p > 0)
    def _():
      def shift(i, _):
        gl_v[pl.ds(i * 16, 16)] = gl_v[pl.ds(i * 16, 16)] + N
        return 0
      lax.fori_loop(0, EPT // 16, shift, 0)

    h = c * CPS + p

    @pl.when(h < NCHUNKS)
    def _():
      pltpu.sync_copy(zacc_hbm.at[pl.ds(s * RPT, RPT)],
                      acc_sh.at[pl.ds(s * RPT, RPT)])
      plsc.subcore_barrier()

      pltpu.async_copy(xflat_hbm.at[gl_v.at[pl.ds(0, GB)]], rows0_v, gsem0)
      pltpu.async_copy(xflat_hbm.at[gl_v.at[pl.ds(GB, GB)]], rows1_v, gsem1)

      def blk_pair(t, _):
        b0 = 2 * t
        pltpu.make_async_copy(
            xflat_hbm.at[pl.ds(0, GB)], rows0_v, gsem0).wait()
        pltpu.sync_copy(rows0_v, acc_sh.at[sl2_v.at[b0]], add=True)

        @pl.when(b0 + 2 < BPP)
        def _():
          pltpu.async_copy(
              xflat_hbm.at[gl_v.at[pl.ds((b0 + 2) * GB, GB)]],
              rows0_v, gsem0)

        pltpu.make_async_copy(
            xflat_hbm.at[pl.ds(0, GB)], rows1_v, gsem1).wait()
        pltpu.sync_copy(rows1_v, acc_sh.at[sl2_v.at[b0 + 1]], add=True)

        @pl.when(b0 + 3 < BPP)
        def _():
          pltpu.async_copy(
              xflat_hbm.at[gl_v.at[pl.ds((b0 + 3) * GB, GB)]],
              rows1_v, gsem1)

        return 0

      lax.fori_loop(0, BPP // 2, blk_pair, 0)
      plsc.subcore_barrier()

      pltpu.sync_copy(acc_sh.at[pl.ds(s * RPT, RPT)],
                      acc_out.at[h, pl.ds(s * RPT, RPT)])
      plsc.subcore_barrier()

    return 0

  lax.fori_loop(0, CPS, one_pass, 0)


def _sc_aggregate(srcp, dstp, typp, xflat):
  zacc = jnp.zeros((ROWS_SC, CW), _f32)

  mesh = plsc.VectorSubcoreMesh(core_axis_name="c", subcore_axis_name="s")
  fn = pl.kernel(
      _sc_body,
      out_type=jax.ShapeDtypeStruct((NCHUNKS, ROWS_SC, CW), _f32),
      mesh=mesh,
      compiler_params=pltpu.CompilerParams(use_tc_tiling_on_sc=False),
      scratch_types=[
          pltpu.VMEM((EPT,), _i32),
          pltpu.VMEM((BPP, GB), _i32),
          pltpu.VMEM((CH,), _i32),
          pltpu.VMEM((CH,), _i32),
          pltpu.VMEM((GB, CW), _f32),
          pltpu.VMEM((GB, CW), _f32),
          pltpu.VMEM_SHARED((ROWS_SC, CW), _f32),
          pltpu.SemaphoreType.DMA,
          pltpu.SemaphoreType.DMA,
      ],
  )
  args = [pltpu.with_memory_space_constraint(a, pltpu.MemorySpace.HBM)
          for a in (srcp, dstp, typp, xflat, zacc)]
  return fn(*args)


NODE_BLK = 200
NODE_BLKS = N // NODE_BLK


def _tc_body(wc_ref, basis_ref, bias_ref, *refs):
  acc_refs = refs[:NCHUNKS - 1]
  cnt_ref = refs[NCHUNKS - 1]
  out_ref = refs[NCHUNKS]
  j = pl.program_id(1)
  w = (wc_ref[j, 0] * basis_ref[0]
       + wc_ref[j, 1] * basis_ref[1]
       + wc_ref[j, 2] * basis_ref[2]
       + wc_ref[j, 3] * basis_ref[3])
  feat = jnp.concatenate([a[0] for a in acc_refs], axis=1)
  deg = cnt_ref[0][:, 0]
  inv = 1.0 / jnp.clip(deg, 1.0, None)
  part = jnp.dot(feat * inv[:, None], w, preferred_element_type=_f32)

  @pl.when(j == 0)
  def _():
    out_ref[...] = part

  @pl.when(j > 0)
  def _():
    out_ref[...] = out_ref[...] + part

  @pl.when(j == R - 1)
  def _():
    out_ref[...] = jnp.maximum(out_ref[...] + bias_ref[...], 0.0)


def _tc_apply(w_comp, basis, h_bias, acc):
  def mk_idx(f):
    return lambda i, j: (f, j * NODE_BLKS + i, 0)

  chunk_specs = [
      pl.BlockSpec((1, NODE_BLK, CW), mk_idx(f)) for f in range(NCHUNKS)
  ]
  return pl.pallas_call(
      _tc_body,
      grid=(NODE_BLKS, R),
      in_specs=[
          pl.BlockSpec(memory_space=pltpu.SMEM),
          pl.BlockSpec((NB_BASES, D, D), lambda i, j: (0, 0, 0)),
          pl.BlockSpec((D,), lambda i, j: (0,)),
          *chunk_specs,
      ],
      out_specs=pl.BlockSpec((NODE_BLK, D), lambda i, j: (i, 0)),
      out_shape=jax.ShapeDtypeStruct((N, D), _f32),
  )(w_comp, basis, h_bias, *([acc] * NCHUNKS))


def kernel(x, edge_index, edge_type, w_comp, basis, h_bias):
  npad = EPAD - E
  src = jnp.concatenate([edge_index[0], jnp.zeros((npad,), _i32)])
  dst = jnp.concatenate([edge_index[1], jnp.zeros((npad,), _i32)])
  typ = jnp.concatenate([edge_type, jnp.full((npad,), R, _i32)])

  xchunks = x.reshape(N, NCHUNKS - 1, CW).transpose(1, 0, 2)
  cnt_chunk = jnp.zeros((1, N, CW), _f32).at[0, :, 0].set(1.0)
  xflat = jnp.concatenate([xchunks, cnt_chunk], 0).reshape(XROWS, CW)

  acc = _sc_aggregate(src, dst, typ, xflat)
  return _tc_apply(w_comp, basis, h_bias, acc)
